# baseline (speedup 1.0000x reference)
import functools

import jax
import jax.numpy as jnp
from jax import lax
from jax.experimental import pallas as pl
from jax.experimental.pallas import tpu as pltpu
from jax.experimental.pallas import tpu_sc as plsc

_T = 1024
_F = 512
_LANES = 16


def _router_body(x_ref, gw_ref, woh_ref, sel_ref):
    n_exp = gw_ref.shape[0]
    scores = lax.dot_general(x_ref[...], gw_ref[...], (((1,), (1,)), ((), ())),
                             preferred_element_type=jnp.float32)
    ids = lax.broadcasted_iota(jnp.int32, scores.shape, 1)
    m1 = jnp.max(scores, axis=-1, keepdims=True)
    i1 = jnp.min(jnp.where(scores == m1, ids, n_exp), axis=-1, keepdims=True)
    s2 = jnp.where(ids == i1, -jnp.inf, scores)
    m2 = jnp.max(s2, axis=-1, keepdims=True)
    i2 = jnp.min(jnp.where(s2 == m2, ids, n_exp), axis=-1, keepdims=True)
    w1 = 1.0 / (1.0 + jnp.exp(m2 - m1))
    woh_ref[...] = jnp.where(ids == i1, w1,
                             jnp.where(ids == i2, 1.0 - w1, 0.0))
    sel_ref[...] = jnp.where((ids == i1) | (ids == i2), 1, 0)


def _router(xf, gate_w):
    ntok, dim = xf.shape
    n_exp = gate_w.shape[0]
    tb = min(512, ntok)
    return pl.pallas_call(
        _router_body,
        grid=(ntok // tb,),
        in_specs=[
            pl.BlockSpec((tb, dim), lambda t: (t, 0)),
            pl.BlockSpec((n_exp, dim), lambda t: (0, 0)),
        ],
        out_specs=[
            pl.BlockSpec((tb, n_exp), lambda t: (t, 0)),
            pl.BlockSpec((tb, n_exp), lambda t: (t, 0)),
        ],
        out_shape=[
            jax.ShapeDtypeStruct((ntok, n_exp), jnp.float32),
            jax.ShapeDtypeStruct((ntok, n_exp), jnp.int32),
        ],
    )(xf, gate_w)


def _sc_gather(xf, idx, total):
    _, dim = xf.shape
    nw = 32
    ch = 16
    rpw = total // nw
    nch = rpw // ch
    mesh = plsc.VectorSubcoreMesh(core_axis_name="c", subcore_axis_name="s")

    @functools.partial(
        pl.kernel, mesh=mesh,
        out_type=jax.ShapeDtypeStruct((total, dim), jnp.float32),
        scratch_types=[
            pltpu.VMEM((rpw,), jnp.int32),
            pltpu.VMEM((ch, dim), jnp.float32),
            pltpu.VMEM((ch, dim), jnp.float32),
            pltpu.SemaphoreType.DMA,
            pltpu.SemaphoreType.DMA,
        ],
    )
    def gk(idx_hbm, x_hbm, out_hbm, idx_v, rows_a, rows_b, sem_a, sem_b):
        wid = lax.axis_index("s") * 2 + lax.axis_index("c")
        base = wid * rpw
        pltpu.sync_copy(idx_hbm.at[pl.ds(base, rpw)], idx_v)

        def start(c, buf, sem):
            cc = jnp.minimum(c, nch - 1)
            return pltpu.async_copy(
                x_hbm.at[idx_v.at[pl.ds(cc * ch, ch)]], buf, sem)

        start(0, rows_a, sem_a)

        def body(i, carry):
            c0 = i * 2
            start(c0 + 1, rows_b, sem_b)
            pltpu.make_async_copy(
                x_hbm.at[idx_v.at[pl.ds(0, ch)]], rows_a, sem_a).wait()
            pltpu.sync_copy(rows_a, out_hbm.at[pl.ds(base + c0 * ch, ch)])
            start(c0 + 2, rows_a, sem_a)
            pltpu.make_async_copy(
                x_hbm.at[idx_v.at[pl.ds(0, ch)]], rows_b, sem_b).wait()
            pltpu.sync_copy(rows_b, out_hbm.at[pl.ds(base + (c0 + 1) * ch, ch)])
            return carry

        lax.fori_loop(0, nch // 2, body, 0)
        pltpu.make_async_copy(
            x_hbm.at[idx_v.at[pl.ds(0, ch)]], rows_a, sem_a).wait()

    return gk(idx, xf)


def _sc_combine(yr, p0, p1, ntok):
    _, dim = yr.shape
    nw = 32
    ch = 16
    tpw = ntok // nw
    nch = tpw // ch
    nv = dim // _LANES
    mesh = plsc.VectorSubcoreMesh(core_axis_name="c", subcore_axis_name="s")

    @functools.partial(
        pl.kernel, mesh=mesh,
        out_type=jax.ShapeDtypeStruct((ntok, dim), jnp.float32),
        scratch_types=[
            pltpu.VMEM((ch,), jnp.int32),
            pltpu.VMEM((ch,), jnp.int32),
            pltpu.VMEM((ch, dim), jnp.float32),
            pltpu.VMEM((ch, dim), jnp.float32),
            pltpu.SemaphoreType.DMA,
            pltpu.SemaphoreType.DMA,
        ],
    )
    def ck(p0_hbm, p1_hbm, yr_hbm, out_hbm, i0_v, i1_v, b0, b1, s0, s1):
        wid = lax.axis_index("s") * 2 + lax.axis_index("c")
        base = wid * tpw

        def chunk(c, carry):
            b = base + c * ch
            pltpu.sync_copy(p0_hbm.at[pl.ds(b, ch)], i0_v)
            pltpu.sync_copy(p1_hbm.at[pl.ds(b, ch)], i1_v)
            cp0 = pltpu.async_copy(yr_hbm.at[i0_v], b0, s0)
            cp1 = pltpu.async_copy(yr_hbm.at[i1_v], b1, s1)
            cp0.wait()
            cp1.wait()
            for r in range(ch):
                def addcol(cc, cr, r=r):
                    sl = pl.ds(cc * _LANES, _LANES)
                    b0[r, sl] = b0[r, sl] + b1[r, sl]
                    return cr
                lax.fori_loop(0, nv, addcol, 0)
            pltpu.sync_copy(b0, out_hbm.at[pl.ds(b, ch)])
            return carry

        lax.fori_loop(0, nch, chunk, 0)

    return ck(p0, p1, yr)


def _ffn_body(eid_ref, xs_ref, wg_ref, wu_ref, wd_ref, w_ref, yr_ref):
    t = pl.program_id(0)
    f = pl.program_id(1)
    nf = pl.num_programs(1)
    e = eid_ref[t]

    @pl.when(f == 0)
    def _zero():
        yr_ref[...] = jnp.zeros_like(yr_ref)

    @pl.when(e >= 0)
    def _compute():
        xsv = xs_ref[...]
        g = lax.dot_general(xsv, wg_ref[0], (((1,), (1,)), ((), ())),
                            preferred_element_type=jnp.float32)
        u = lax.dot_general(xsv, wu_ref[0], (((1,), (1,)), ((), ())),
                            preferred_element_type=jnp.float32)
        h = g * (1.0 / (1.0 + jnp.exp(-g))) * u
        d = lax.dot_general(h, wd_ref[0], (((1,), (1,)), ((), ())),
                            preferred_element_type=jnp.float32)
        yr_ref[...] += d

    @pl.when((f == nf - 1) & (e >= 0))
    def _scale():
        yr_ref[...] *= w_ref[0]


def _tc_ffn(xs, Wg, Wu, Wd, w3, tile_eid):
    total, dim = xs.shape
    n_exp, dff, _ = Wg.shape
    nt = total // _T
    nf = dff // _F
    gridspec = pltpu.PrefetchScalarGridSpec(
        num_scalar_prefetch=1,
        grid=(nt, nf),
        in_specs=[
            pl.BlockSpec((_T, dim), lambda t, f, eid: (t, 0)),
            pl.BlockSpec((1, _F, dim),
                         lambda t, f, eid: (jnp.maximum(eid[t], 0), f, 0)),
            pl.BlockSpec((1, _F, dim),
                         lambda t, f, eid: (jnp.maximum(eid[t], 0), f, 0)),
            pl.BlockSpec((1, dim, _F),
                         lambda t, f, eid: (jnp.maximum(eid[t], 0), 0, f)),
            pl.BlockSpec((1, _T, 1), lambda t, f, eid: (t, 0, 0)),
        ],
        out_specs=pl.BlockSpec((_T, dim), lambda t, f, eid: (t, 0)),
    )
    return pl.pallas_call(
        _ffn_body,
        grid_spec=gridspec,
        out_shape=jax.ShapeDtypeStruct((total, dim), jnp.float32),
        compiler_params=pltpu.CompilerParams(
            vmem_limit_bytes=100 * 1024 * 1024),
    )(tile_eid, xs, Wg, Wu, Wd, w3)


def kernel(x, gate_w, Wg, Wu, Wd):
    orig_shape = x.shape
    xf = x.reshape(-1, x.shape[-1])
    ntok, dim = xf.shape
    n_exp = gate_w.shape[0]
    topk = 2
    n_assign = ntok * topk
    nt = n_assign // _T + n_exp
    total = nt * _T

    woh, sel = _router(xf, gate_w)

    selb = sel > 0
    ranks = jnp.cumsum(sel, axis=0) - sel
    cnt = jnp.sum(sel, axis=0)
    pad_cnt = ((cnt + _T - 1) // _T) * _T
    seg_end = jnp.cumsum(pad_cnt)
    off = seg_end - pad_cnt
    pos = off[None, :] + ranks
    posd = jnp.where(selb, pos, total).astype(jnp.int32).reshape(-1)
    tok_ids = jnp.broadcast_to(
        jnp.arange(ntok, dtype=jnp.int32)[:, None], (ntok, n_exp)).reshape(-1)
    tok_sorted = jnp.zeros((total + 8,), jnp.int32).at[posd].set(tok_ids)[:total]
    w_sorted = jnp.zeros((total + 8,), jnp.float32).at[posd].set(
        woh.reshape(-1))[:total]
    tile_start = jnp.arange(nt, dtype=jnp.int32) * _T
    eid = jnp.sum((tile_start[:, None] >= seg_end[None, :]).astype(jnp.int32),
                  axis=1)
    tile_eid = jnp.where(tile_start < seg_end[-1], eid, -1).astype(jnp.int32)
    big = jnp.int32(1 << 30)
    pm = jnp.where(selb, pos, big)
    p0 = jnp.min(pm, axis=1).astype(jnp.int32)
    p1 = (jnp.sum(jnp.where(selb, pos, 0), axis=1)
          - jnp.min(pm, axis=1)).astype(jnp.int32)

    xs = _sc_gather(xf, tok_sorted, total)
    yr = _tc_ffn(xs, Wg, Wu, Wd, w_sorted.reshape(nt, _T, 1), tile_eid)
    y = _sc_combine(yr, p0, p1, ntok)
    return y.reshape(orig_shape)


# speedup vs baseline: 1.0739x; 1.0739x over previous
import functools

import jax
import jax.numpy as jnp
from jax import lax
from jax.experimental import pallas as pl
from jax.experimental.pallas import tpu as pltpu
from jax.experimental.pallas import tpu_sc as plsc

_T = 512
_F = 512
_LANES = 16


def _router_body(x_ref, gw_ref, woh_ref, sel_ref):
    n_exp = gw_ref.shape[0]
    scores = lax.dot_general(x_ref[...], gw_ref[...], (((1,), (1,)), ((), ())),
                             preferred_element_type=jnp.float32)
    ids = lax.broadcasted_iota(jnp.int32, scores.shape, 1)
    m1 = jnp.max(scores, axis=-1, keepdims=True)
    i1 = jnp.min(jnp.where(scores == m1, ids, n_exp), axis=-1, keepdims=True)
    s2 = jnp.where(ids == i1, -jnp.inf, scores)
    m2 = jnp.max(s2, axis=-1, keepdims=True)
    i2 = jnp.min(jnp.where(s2 == m2, ids, n_exp), axis=-1, keepdims=True)
    w1 = 1.0 / (1.0 + jnp.exp(m2 - m1))
    woh_ref[...] = jnp.where(ids == i1, w1,
                             jnp.where(ids == i2, 1.0 - w1, 0.0))
    sel_ref[...] = jnp.where((ids == i1) | (ids == i2), 1, 0)


def _router(xf, gate_w):
    ntok, dim = xf.shape
    n_exp = gate_w.shape[0]
    tb = min(512, ntok)
    return pl.pallas_call(
        _router_body,
        grid=(ntok // tb,),
        in_specs=[
            pl.BlockSpec((tb, dim), lambda t: (t, 0)),
            pl.BlockSpec((n_exp, dim), lambda t: (0, 0)),
        ],
        out_specs=[
            pl.BlockSpec((tb, n_exp), lambda t: (t, 0)),
            pl.BlockSpec((tb, n_exp), lambda t: (t, 0)),
        ],
        out_shape=[
            jax.ShapeDtypeStruct((ntok, n_exp), jnp.float32),
            jax.ShapeDtypeStruct((ntok, n_exp), jnp.int32),
        ],
    )(xf, gate_w)


def _sc_gather(xf, idx, total):
    _, dim = xf.shape
    nw = 32
    ch = 16
    rpw = total // nw
    nch = rpw // ch
    mesh = plsc.VectorSubcoreMesh(core_axis_name="c", subcore_axis_name="s")

    @functools.partial(
        pl.kernel, mesh=mesh,
        out_type=jax.ShapeDtypeStruct((total, dim), jnp.float32),
        scratch_types=[
            pltpu.VMEM((rpw,), jnp.int32),
            pltpu.VMEM((ch, dim), jnp.float32),
            pltpu.VMEM((ch, dim), jnp.float32),
            pltpu.SemaphoreType.DMA,
            pltpu.SemaphoreType.DMA,
        ],
    )
    def gk(idx_hbm, x_hbm, out_hbm, idx_v, rows_a, rows_b, sem_a, sem_b):
        wid = lax.axis_index("s") * 2 + lax.axis_index("c")
        base = wid * rpw
        pltpu.sync_copy(idx_hbm.at[pl.ds(base, rpw)], idx_v)

        def start(c, buf, sem):
            cc = jnp.minimum(c, nch - 1)
            return pltpu.async_copy(
                x_hbm.at[idx_v.at[pl.ds(cc * ch, ch)]], buf, sem)

        start(0, rows_a, sem_a)

        def body(i, carry):
            c0 = i * 2
            start(c0 + 1, rows_b, sem_b)
            pltpu.make_async_copy(
                x_hbm.at[idx_v.at[pl.ds(0, ch)]], rows_a, sem_a).wait()
            pltpu.sync_copy(rows_a, out_hbm.at[pl.ds(base + c0 * ch, ch)])
            start(c0 + 2, rows_a, sem_a)
            pltpu.make_async_copy(
                x_hbm.at[idx_v.at[pl.ds(0, ch)]], rows_b, sem_b).wait()
            pltpu.sync_copy(rows_b, out_hbm.at[pl.ds(base + (c0 + 1) * ch, ch)])
            return carry

        lax.fori_loop(0, nch // 2, body, 0)
        pltpu.make_async_copy(
            x_hbm.at[idx_v.at[pl.ds(0, ch)]], rows_a, sem_a).wait()

    return gk(idx, xf)


def _sc_combine(yr, p0, p1, ntok):
    _, dim = yr.shape
    nw = 32
    ch = 16
    tpw = ntok // nw
    nch = tpw // ch
    nv = dim // _LANES
    mesh = plsc.VectorSubcoreMesh(core_axis_name="c", subcore_axis_name="s")

    @functools.partial(
        pl.kernel, mesh=mesh,
        out_type=jax.ShapeDtypeStruct((ntok, dim), jnp.float32),
        scratch_types=[
            pltpu.VMEM((ch,), jnp.int32),
            pltpu.VMEM((ch,), jnp.int32),
            pltpu.VMEM((ch, dim), jnp.float32),
            pltpu.VMEM((ch, dim), jnp.float32),
            pltpu.SemaphoreType.DMA,
            pltpu.SemaphoreType.DMA,
        ],
    )
    def ck(p0_hbm, p1_hbm, yr_hbm, out_hbm, i0_v, i1_v, b0, b1, s0, s1):
        wid = lax.axis_index("s") * 2 + lax.axis_index("c")
        base = wid * tpw

        def chunk(c, carry):
            b = base + c * ch
            pltpu.sync_copy(p0_hbm.at[pl.ds(b, ch)], i0_v)
            pltpu.sync_copy(p1_hbm.at[pl.ds(b, ch)], i1_v)
            cp0 = pltpu.async_copy(yr_hbm.at[i0_v], b0, s0)
            cp1 = pltpu.async_copy(yr_hbm.at[i1_v], b1, s1)
            cp0.wait()
            cp1.wait()
            for r in range(ch):
                def addcol(cc, cr, r=r):
                    sl = pl.ds(cc * _LANES, _LANES)
                    b0[r, sl] = b0[r, sl] + b1[r, sl]
                    return cr
                lax.fori_loop(0, nv, addcol, 0)
            pltpu.sync_copy(b0, out_hbm.at[pl.ds(b, ch)])
            return carry

        lax.fori_loop(0, nch, chunk, 0)

    return ck(p0, p1, yr)


def _ffn_body(eid_ref, xs_ref, wg_ref, wu_ref, wd_ref, w_ref, yr_ref):
    t = pl.program_id(0)
    f = pl.program_id(1)
    nf = pl.num_programs(1)
    e = eid_ref[t]

    @pl.when(f == 0)
    def _zero():
        yr_ref[...] = jnp.zeros_like(yr_ref)

    @pl.when(e >= 0)
    def _compute():
        xsv = xs_ref[...]
        g = lax.dot_general(xsv, wg_ref[0], (((1,), (1,)), ((), ())),
                            preferred_element_type=jnp.float32)
        u = lax.dot_general(xsv, wu_ref[0], (((1,), (1,)), ((), ())),
                            preferred_element_type=jnp.float32)
        h = g * (1.0 / (1.0 + jnp.exp(-g))) * u
        d = lax.dot_general(h, wd_ref[0], (((1,), (1,)), ((), ())),
                            preferred_element_type=jnp.float32)
        yr_ref[...] += d

    @pl.when((f == nf - 1) & (e >= 0))
    def _scale():
        yr_ref[...] *= w_ref[0]


def _tc_ffn(xs, Wg, Wu, Wd, w3, tile_eid):
    total, dim = xs.shape
    n_exp, dff, _ = Wg.shape
    nt = total // _T
    nf = dff // _F
    gridspec = pltpu.PrefetchScalarGridSpec(
        num_scalar_prefetch=1,
        grid=(nt, nf),
        in_specs=[
            pl.BlockSpec((_T, dim), lambda t, f, eid: (t, 0)),
            pl.BlockSpec((1, _F, dim),
                         lambda t, f, eid: (jnp.maximum(eid[t], 0), f, 0)),
            pl.BlockSpec((1, _F, dim),
                         lambda t, f, eid: (jnp.maximum(eid[t], 0), f, 0)),
            pl.BlockSpec((1, dim, _F),
                         lambda t, f, eid: (jnp.maximum(eid[t], 0), 0, f)),
            pl.BlockSpec((1, _T, 1), lambda t, f, eid: (t, 0, 0)),
        ],
        out_specs=pl.BlockSpec((_T, dim), lambda t, f, eid: (t, 0)),
    )
    return pl.pallas_call(
        _ffn_body,
        grid_spec=gridspec,
        out_shape=jax.ShapeDtypeStruct((total, dim), jnp.float32),
        compiler_params=pltpu.CompilerParams(
            vmem_limit_bytes=100 * 1024 * 1024),
    )(tile_eid, xs, Wg, Wu, Wd, w3)


def kernel(x, gate_w, Wg, Wu, Wd):
    orig_shape = x.shape
    xf = x.reshape(-1, x.shape[-1])
    ntok, dim = xf.shape
    n_exp = gate_w.shape[0]
    topk = 2
    n_assign = ntok * topk
    nt = n_assign // _T + n_exp
    total = nt * _T

    woh, sel = _router(xf, gate_w)

    selb = sel > 0
    ranks = jnp.cumsum(sel, axis=0) - sel
    cnt = jnp.sum(sel, axis=0)
    pad_cnt = ((cnt + _T - 1) // _T) * _T
    seg_end = jnp.cumsum(pad_cnt)
    off = seg_end - pad_cnt
    pos = off[None, :] + ranks
    posd = jnp.where(selb, pos, total).astype(jnp.int32).reshape(-1)
    tok_ids = jnp.broadcast_to(
        jnp.arange(ntok, dtype=jnp.int32)[:, None], (ntok, n_exp)).reshape(-1)
    tok_sorted = jnp.zeros((total + 8,), jnp.int32).at[posd].set(tok_ids)[:total]
    w_sorted = jnp.zeros((total + 8,), jnp.float32).at[posd].set(
        woh.reshape(-1))[:total]
    tile_start = jnp.arange(nt, dtype=jnp.int32) * _T
    eid = jnp.sum((tile_start[:, None] >= seg_end[None, :]).astype(jnp.int32),
                  axis=1)
    tile_eid = jnp.where(tile_start < seg_end[-1], eid, -1).astype(jnp.int32)
    big = jnp.int32(1 << 30)
    pm = jnp.where(selb, pos, big)
    p0 = jnp.min(pm, axis=1).astype(jnp.int32)
    p1 = (jnp.sum(jnp.where(selb, pos, 0), axis=1)
          - jnp.min(pm, axis=1)).astype(jnp.int32)

    xs = _sc_gather(xf, tok_sorted, total)
    yr = _tc_ffn(xs, Wg, Wu, Wd, w_sorted.reshape(nt, _T, 1), tile_eid)
    y = _sc_combine(yr, p0, p1, ntok)
    return y.reshape(orig_shape)


# speedup vs baseline: 1.2218x; 1.1377x over previous
import functools

import jax
import jax.numpy as jnp
from jax import lax
from jax.experimental import pallas as pl
from jax.experimental.pallas import tpu as pltpu
from jax.experimental.pallas import tpu_sc as plsc

_T = 512
_F = 512
_LANES = 16


def _router_body(x_ref, gw_ref, woh_ref, sel_ref):
    n_exp = gw_ref.shape[0]
    scores = lax.dot_general(x_ref[...], gw_ref[...], (((1,), (1,)), ((), ())),
                             preferred_element_type=jnp.float32)
    ids = lax.broadcasted_iota(jnp.int32, scores.shape, 1)
    m1 = jnp.max(scores, axis=-1, keepdims=True)
    i1 = jnp.min(jnp.where(scores == m1, ids, n_exp), axis=-1, keepdims=True)
    s2 = jnp.where(ids == i1, -jnp.inf, scores)
    m2 = jnp.max(s2, axis=-1, keepdims=True)
    i2 = jnp.min(jnp.where(s2 == m2, ids, n_exp), axis=-1, keepdims=True)
    w1 = 1.0 / (1.0 + jnp.exp(m2 - m1))
    woh_ref[...] = jnp.where(ids == i1, w1,
                             jnp.where(ids == i2, 1.0 - w1, 0.0))
    sel_ref[...] = jnp.where((ids == i1) | (ids == i2), 1, 0)


def _router(xf, gate_w):
    ntok, dim = xf.shape
    n_exp = gate_w.shape[0]
    tb = min(512, ntok)
    return pl.pallas_call(
        _router_body,
        grid=(ntok // tb,),
        in_specs=[
            pl.BlockSpec((tb, dim), lambda t: (t, 0)),
            pl.BlockSpec((n_exp, dim), lambda t: (0, 0)),
        ],
        out_specs=[
            pl.BlockSpec((tb, n_exp), lambda t: (t, 0)),
            pl.BlockSpec((tb, n_exp), lambda t: (t, 0)),
        ],
        out_shape=[
            jax.ShapeDtypeStruct((ntok, n_exp), jnp.float32),
            jax.ShapeDtypeStruct((ntok, n_exp), jnp.int32),
        ],
    )(xf, gate_w)


def _sc_gather(xf, idx, total):
    _, dim = xf.shape
    nw = 32
    ch = 16
    rpw = total // nw
    nch = rpw // ch
    mesh = plsc.VectorSubcoreMesh(core_axis_name="c", subcore_axis_name="s")

    @functools.partial(
        pl.kernel, mesh=mesh,
        out_type=jax.ShapeDtypeStruct((total, dim), jnp.float32),
        scratch_types=[
            pltpu.VMEM((rpw,), jnp.int32),
            pltpu.VMEM((ch, dim), jnp.float32),
            pltpu.VMEM((ch, dim), jnp.float32),
            pltpu.SemaphoreType.DMA,
            pltpu.SemaphoreType.DMA,
        ],
    )
    def gk(idx_hbm, x_hbm, out_hbm, idx_v, rows_a, rows_b, sem_a, sem_b):
        wid = lax.axis_index("s") * 2 + lax.axis_index("c")
        base = wid * rpw
        pltpu.sync_copy(idx_hbm.at[pl.ds(base, rpw)], idx_v)

        def start(c, buf, sem):
            cc = jnp.minimum(c, nch - 1)
            return pltpu.async_copy(
                x_hbm.at[idx_v.at[pl.ds(cc * ch, ch)]], buf, sem)

        start(0, rows_a, sem_a)

        def body(i, carry):
            c0 = i * 2
            start(c0 + 1, rows_b, sem_b)
            pltpu.make_async_copy(
                x_hbm.at[idx_v.at[pl.ds(0, ch)]], rows_a, sem_a).wait()
            pltpu.sync_copy(rows_a, out_hbm.at[pl.ds(base + c0 * ch, ch)])
            start(c0 + 2, rows_a, sem_a)
            pltpu.make_async_copy(
                x_hbm.at[idx_v.at[pl.ds(0, ch)]], rows_b, sem_b).wait()
            pltpu.sync_copy(rows_b, out_hbm.at[pl.ds(base + (c0 + 1) * ch, ch)])
            return carry

        lax.fori_loop(0, nch // 2, body, 0)
        pltpu.make_async_copy(
            x_hbm.at[idx_v.at[pl.ds(0, ch)]], rows_a, sem_a).wait()

    return gk(idx, xf)


def _sc_combine(yr, p0, p1, ntok):
    _, dim = yr.shape
    nw = 32
    ch = 16
    tpw = ntok // nw
    nch = tpw // ch
    nv = dim // _LANES
    mesh = plsc.VectorSubcoreMesh(core_axis_name="c", subcore_axis_name="s")

    @functools.partial(
        pl.kernel, mesh=mesh,
        out_type=jax.ShapeDtypeStruct((ntok, dim), jnp.float32),
        scratch_types=[
            pltpu.VMEM((ch,), jnp.int32),
            pltpu.VMEM((ch,), jnp.int32),
            pltpu.VMEM((ch, dim), jnp.float32),
            pltpu.VMEM((ch, dim), jnp.float32),
            pltpu.SemaphoreType.DMA,
            pltpu.SemaphoreType.DMA,
        ],
    )
    def ck(p0_hbm, p1_hbm, yr_hbm, out_hbm, i0_v, i1_v, b0, b1, s0, s1):
        wid = lax.axis_index("s") * 2 + lax.axis_index("c")
        base = wid * tpw

        def chunk(c, carry):
            b = base + c * ch
            pltpu.sync_copy(p0_hbm.at[pl.ds(b, ch)], i0_v)
            pltpu.sync_copy(p1_hbm.at[pl.ds(b, ch)], i1_v)
            cp0 = pltpu.async_copy(yr_hbm.at[i0_v], b0, s0)
            cp1 = pltpu.async_copy(yr_hbm.at[i1_v], b1, s1)
            cp0.wait()
            cp1.wait()
            for r in range(ch):
                def addcol(cc, cr, r=r):
                    sl = pl.ds(cc * _LANES, _LANES)
                    b0[r, sl] = b0[r, sl] + b1[r, sl]
                    return cr
                lax.fori_loop(0, nv, addcol, 0)
            pltpu.sync_copy(b0, out_hbm.at[pl.ds(b, ch)])
            return carry

        lax.fori_loop(0, nch, chunk, 0)

    return ck(p0, p1, yr)


def _ffn_body(eid_ref, xs_ref, wg_ref, wu_ref, wd_ref, w_ref, yr_ref):
    t = pl.program_id(0)
    f = pl.program_id(1)
    nf = pl.num_programs(1)
    e = eid_ref[t]

    @pl.when(f == 0)
    def _zero():
        yr_ref[...] = jnp.zeros_like(yr_ref)

    @pl.when(e >= 0)
    def _compute():
        xsv = xs_ref[...]
        g = lax.dot_general(xsv, wg_ref[0], (((1,), (1,)), ((), ())),
                            preferred_element_type=jnp.float32)
        u = lax.dot_general(xsv, wu_ref[0], (((1,), (1,)), ((), ())),
                            preferred_element_type=jnp.float32)
        h = g * (1.0 / (1.0 + jnp.exp(-g))) * u
        d = lax.dot_general(h, wd_ref[0], (((1,), (1,)), ((), ())),
                            preferred_element_type=jnp.float32)
        yr_ref[...] += d

    @pl.when((f == nf - 1) & (e >= 0))
    def _scale():
        yr_ref[...] *= w_ref[0]


def _tc_ffn(xs, Wg, Wu, Wd, w3, tile_eid):
    total, dim = xs.shape
    n_exp, dff, _ = Wg.shape
    nt = total // _T
    nf = dff // _F
    gridspec = pltpu.PrefetchScalarGridSpec(
        num_scalar_prefetch=1,
        grid=(nt, nf),
        in_specs=[
            pl.BlockSpec((_T, dim), lambda t, f, eid: (t, 0)),
            pl.BlockSpec((1, _F, dim),
                         lambda t, f, eid: (jnp.maximum(eid[t], 0),
                                            jnp.where(eid[t] < 0, 0, f), 0)),
            pl.BlockSpec((1, _F, dim),
                         lambda t, f, eid: (jnp.maximum(eid[t], 0),
                                            jnp.where(eid[t] < 0, 0, f), 0)),
            pl.BlockSpec((1, dim, _F),
                         lambda t, f, eid: (jnp.maximum(eid[t], 0), 0,
                                            jnp.where(eid[t] < 0, 0, f))),
            pl.BlockSpec((1, _T, 1), lambda t, f, eid: (t, 0, 0)),
        ],
        out_specs=pl.BlockSpec((_T, dim), lambda t, f, eid: (t, 0)),
    )
    return pl.pallas_call(
        _ffn_body,
        grid_spec=gridspec,
        out_shape=jax.ShapeDtypeStruct((total, dim), jnp.float32),
        compiler_params=pltpu.CompilerParams(
            vmem_limit_bytes=100 * 1024 * 1024),
    )(tile_eid, xs, Wg, Wu, Wd, w3)


def kernel(x, gate_w, Wg, Wu, Wd):
    orig_shape = x.shape
    xf = x.reshape(-1, x.shape[-1])
    ntok, dim = xf.shape
    n_exp = gate_w.shape[0]
    topk = 2
    n_assign = ntok * topk
    nt = n_assign // _T + n_exp
    total = nt * _T

    woh, sel = _router(xf, gate_w)

    selb = sel > 0
    ranks = jnp.cumsum(sel, axis=0) - sel
    cnt = jnp.sum(sel, axis=0)
    pad_cnt = ((cnt + _T - 1) // _T) * _T
    seg_end = jnp.cumsum(pad_cnt)
    off = seg_end - pad_cnt
    pos = off[None, :] + ranks
    posd = jnp.where(selb, pos, total).astype(jnp.int32).reshape(-1)
    tok_ids = jnp.broadcast_to(
        jnp.arange(ntok, dtype=jnp.int32)[:, None], (ntok, n_exp)).reshape(-1)
    tok_sorted = jnp.zeros((total + 8,), jnp.int32).at[posd].set(tok_ids)[:total]
    w_sorted = jnp.zeros((total + 8,), jnp.float32).at[posd].set(
        woh.reshape(-1))[:total]
    tile_start = jnp.arange(nt, dtype=jnp.int32) * _T
    eid = jnp.sum((tile_start[:, None] >= seg_end[None, :]).astype(jnp.int32),
                  axis=1)
    tile_eid = jnp.where(tile_start < seg_end[-1], eid, -1).astype(jnp.int32)
    big = jnp.int32(1 << 30)
    pm = jnp.where(selb, pos, big)
    p0 = jnp.min(pm, axis=1).astype(jnp.int32)
    p1 = (jnp.sum(jnp.where(selb, pos, 0), axis=1)
          - jnp.min(pm, axis=1)).astype(jnp.int32)

    xs = _sc_gather(xf, tok_sorted, total)
    yr = _tc_ffn(xs, Wg, Wu, Wd, w_sorted.reshape(nt, _T, 1), tile_eid)
    y = _sc_combine(yr, p0, p1, ntok)
    return y.reshape(orig_shape)
